# baseline (speedup 1.0000x reference)
import functools

import jax
import jax.numpy as jnp
from jax import lax
from jax.experimental import pallas as pl
from jax.experimental.pallas import tpu as pltpu
from jax.experimental.pallas import tpu_sc as plsc

NUM_GENRES = 20
EMBED_DIM = 240
TITLE_DIM = 768
MOVIE_FEAT_DIM = 64
BATCH = 16384

_NC = 2
_NS = 16
_NW = _NC * _NS
_B_PER_W = BATCH // _NW


def _norm_table_body(tab_ref, out_ref):
    t = tab_ref[...]
    norm = jnp.sqrt(jnp.sum(t * t, axis=1, keepdims=True))
    out_ref[...] = t / jnp.maximum(norm, 1e-12)


def _normalize_table(user_table):
    return pl.pallas_call(
        _norm_table_body,
        out_shape=jax.ShapeDtypeStruct((NUM_GENRES, EMBED_DIM), jnp.float32),
    )(user_table)


def _sc_gather_body(idx_hbm, table_hbm, out_hbm, idx_v, rows_v, sem):
    wid = lax.axis_index("s") * _NC + lax.axis_index("c")
    base = wid * _B_PER_W
    pltpu.sync_copy(idx_hbm.at[pl.ds(base, _B_PER_W)], idx_v)
    pltpu.async_copy(table_hbm.at[idx_v], rows_v, sem).wait()
    pltpu.sync_copy(rows_v, out_hbm.at[pl.ds(base, _B_PER_W)])


_sc_gather = functools.partial(
    pl.kernel,
    out_type=jax.ShapeDtypeStruct((BATCH, EMBED_DIM), jnp.float32),
    mesh=plsc.VectorSubcoreMesh(core_axis_name="c", subcore_axis_name="s"),
    scratch_types=[
        pltpu.VMEM((_B_PER_W,), jnp.int32),
        pltpu.VMEM((_B_PER_W, EMBED_DIM), jnp.float32),
        pltpu.SemaphoreType.DMA,
    ],
    compiler_params=pltpu.CompilerParams(use_tc_tiling_on_sc=False),
)(_sc_gather_body)


_BM = 1024


def _movie_body(title_ref, feat_ref, wt_ref, wm_ref, b_ref, out_ref):
    acc = jnp.dot(title_ref[...], wt_ref[...], preferred_element_type=jnp.float32)
    acc = acc + jnp.dot(feat_ref[...], wm_ref[...], preferred_element_type=jnp.float32)
    acc = acc + b_ref[...]
    norm = jnp.sqrt(jnp.sum(acc * acc, axis=1, keepdims=True))
    out_ref[...] = acc / jnp.maximum(norm, 1e-12)


def _movie_tower(title_embeddings, movie_features, W_movie, b_movie):
    w_t = W_movie[:TITLE_DIM]
    w_m = W_movie[TITLE_DIM:]
    bias = b_movie.reshape(1, EMBED_DIM)
    grid = (BATCH // _BM,)
    return pl.pallas_call(
        _movie_body,
        grid=grid,
        in_specs=[
            pl.BlockSpec((_BM, TITLE_DIM), lambda i: (i, 0)),
            pl.BlockSpec((_BM, MOVIE_FEAT_DIM), lambda i: (i, 0)),
            pl.BlockSpec((TITLE_DIM, EMBED_DIM), lambda i: (0, 0)),
            pl.BlockSpec((MOVIE_FEAT_DIM, EMBED_DIM), lambda i: (0, 0)),
            pl.BlockSpec((1, EMBED_DIM), lambda i: (0, 0)),
        ],
        out_specs=pl.BlockSpec((_BM, EMBED_DIM), lambda i: (i, 0)),
        out_shape=jax.ShapeDtypeStruct((BATCH, EMBED_DIM), jnp.float32),
    )(title_embeddings, movie_features, w_t, w_m, bias)


def kernel(user_features, title_embeddings, movie_features, user_table, W_movie, b_movie):
    norm_table = _normalize_table(user_table)
    user_embedding = _sc_gather(user_features, norm_table)
    movie_embedding = _movie_tower(title_embeddings, movie_features, W_movie, b_movie)
    return (user_embedding, movie_embedding)
